# baseline (speedup 1.0000x reference)
import functools

import jax
import jax.numpy as jnp
from jax import lax
from jax.experimental import pallas as pl
from jax.experimental.pallas import tpu as pltpu
from jax.experimental.pallas import tpu_sc as plsc

D = 32
NC, NS = 2, 16
NW = NC * NS
BW = 128
L = 16


def _make_gather(T, B, V):
    assert B == NW * BW
    mesh = plsc.VectorSubcoreMesh(core_axis_name="c", subcore_axis_name="s")

    @functools.partial(
        pl.kernel,
        out_type=jax.ShapeDtypeStruct((T, D, B), jnp.float32),
        mesh=mesh,
        scratch_types=[
            pltpu.VMEM((T, BW), jnp.int32),
            pltpu.VMEM((BW, D), jnp.float32),
            pltpu.VMEM((BW, D), jnp.float32),
            pltpu.VMEM((D, BW), jnp.float32),
            pltpu.VMEM((D, BW), jnp.float32),
            pltpu.SemaphoreType.DMA,
            pltpu.SemaphoreType.DMA,
            pltpu.SemaphoreType.DMA,
            pltpu.SemaphoreType.DMA,
        ],
        compiler_params=pltpu.CompilerParams(
            use_tc_tiling_on_sc=False, needs_layout_passes=False),
    )
    def gather(table_hbm, xt_hbm, q_hbm, idx_v, rows_v0, rows_v1,
               qt_v0, qt_v1, sem_i, sem_g, sem_o0, sem_o1):
        rows_v = (rows_v0, rows_v1)
        qt_v = (qt_v0, qt_v1)
        wid = lax.axis_index("s") * NC + lax.axis_index("c")
        b0 = wid * BW
        sem_o = (sem_o0, sem_o1)

        pltpu.sync_copy(xt_hbm.at[:, pl.ds(b0, BW)], idx_v)

        bvecs = [lax.iota(jnp.int32, L) + (g * L) for g in range(BW // L)]
        cvecs = [jnp.full((L,), c, jnp.int32) for c in range(D)]

        def fire(t, s):
            return pltpu.async_copy(
                table_hbm.at[idx_v.at[t]], rows_v[s], sem_g)

        def transpose(s):
            src = rows_v[s]
            dst = qt_v[s]
            for c in range(D):
                for g in range(BW // L):
                    vals = plsc.load_gather(src, [bvecs[g], cvecs[c]])
                    dst[c, pl.ds(g * L, L)] = vals

        def store_q(t, s):
            pltpu.async_copy(
                qt_v[s], q_hbm.at[t].at[:, pl.ds(b0, BW)], sem_o[s])

        def wait_store(s):
            pltpu.make_async_copy(
                qt_v[s], q_hbm.at[0].at[:, pl.ds(b0, BW)], sem_o[s]).wait()

        cp0 = fire(0, 0)
        cp1 = fire(1, 1)

        for s, cp in ((0, cp0), (1, cp1)):
            cp.wait()
            transpose(s)
            fire(s + 2, s)
            store_q(s, s)

        @pl.loop(2, T - 2, step=2)
        def _(t0):
            for s in (0, 1):
                t = t0 + s
                pltpu.make_async_copy(
                    table_hbm.at[idx_v.at[0]], rows_v[s], sem_g).wait()
                transpose(s)
                fire(t + 2, s)
                wait_store(s)
                store_q(t, s)

        for s in (0, 1):
            t = T - 2 + s
            pltpu.make_async_copy(
                table_hbm.at[idx_v.at[0]], rows_v[s], sem_g).wait()
            transpose(s)
            wait_store(s)
            store_q(t, s)

        wait_store(0)
        wait_store(1)

    return gather


def kernel(X, table):
    Bb, T = X.shape
    V, d = table.shape
    Xt = X.T.astype(jnp.int32)
    Q = _make_gather(T, Bb, V)(table, Xt)
    return Q.transpose(2, 0, 1)


# speedup vs baseline: 1.4093x; 1.4093x over previous
import functools

import jax
import jax.numpy as jnp
from jax import lax
from jax.experimental import pallas as pl
from jax.experimental.pallas import tpu as pltpu
from jax.experimental.pallas import tpu_sc as plsc

D = 32
NC, NS = 2, 16
NW = NC * NS
BW = 128
L = 16


def _make_gather(T, B, V):
    assert B == NW * BW
    mesh = plsc.VectorSubcoreMesh(core_axis_name="c", subcore_axis_name="s")

    @functools.partial(
        pl.kernel,
        out_type=jax.ShapeDtypeStruct((T, D, B), jnp.float32),
        mesh=mesh,
        scratch_types=[
            pltpu.VMEM((T, BW), jnp.int32),
            pltpu.VMEM((BW, D), jnp.float32),
            pltpu.VMEM((BW, D), jnp.float32),
            pltpu.VMEM((D, BW), jnp.float32),
            pltpu.VMEM((D, BW), jnp.float32),
            pltpu.SemaphoreType.DMA,
            pltpu.SemaphoreType.DMA,
            pltpu.SemaphoreType.DMA,
            pltpu.SemaphoreType.DMA,
        ],
        compiler_params=pltpu.CompilerParams(
            use_tc_tiling_on_sc=False, needs_layout_passes=False),
    )
    def gather(table_hbm, xt_hbm, q_hbm, idx_v, rows_v0, rows_v1,
               qt_v0, qt_v1, sem_i, sem_g, sem_o0, sem_o1):
        rows_v = (rows_v0, rows_v1)
        qt_v = (qt_v0, qt_v1)
        wid = lax.axis_index("s") * NC + lax.axis_index("c")
        b0 = wid * BW
        sem_o = (sem_o0, sem_o1)

        pltpu.sync_copy(xt_hbm.at[:, pl.ds(b0, BW)], idx_v)

        bvecs = [lax.iota(jnp.int32, L) + (g * L) for g in range(BW // L)]

        def fire(t, s):
            return pltpu.async_copy(
                table_hbm.at[idx_v.at[t]], rows_v[s], sem_g)

        def transpose(s):
            src = rows_v[s]
            dst = qt_v[s]

            @plsc.parallel_loop(0, D, unroll=4)
            def _(c):
                cvec = lax.broadcast(c, (L,))
                for g in range(BW // L):
                    vals = plsc.load_gather(src, [bvecs[g], cvec])
                    dst[c, pl.ds(g * L, L)] = vals

        def store_q(t, s):
            pltpu.async_copy(
                qt_v[s], q_hbm.at[t].at[:, pl.ds(b0, BW)], sem_o[s])

        def wait_store(s):
            pltpu.make_async_copy(
                qt_v[s], q_hbm.at[0].at[:, pl.ds(b0, BW)], sem_o[s]).wait()

        cp0 = fire(0, 0)
        cp1 = fire(1, 1)

        for s, cp in ((0, cp0), (1, cp1)):
            cp.wait()
            transpose(s)
            fire(s + 2, s)
            store_q(s, s)

        @pl.loop(2, T - 2, step=2)
        def _(t0):
            for s in (0, 1):
                t = t0 + s
                pltpu.make_async_copy(
                    table_hbm.at[idx_v.at[0]], rows_v[s], sem_g).wait()
                transpose(s)
                fire(t + 2, s)
                wait_store(s)
                store_q(t, s)

        for s in (0, 1):
            t = T - 2 + s
            pltpu.make_async_copy(
                table_hbm.at[idx_v.at[0]], rows_v[s], sem_g).wait()
            transpose(s)
            wait_store(s)
            store_q(t, s)

        wait_store(0)
        wait_store(1)

    return gather


def kernel(X, table):
    Bb, T = X.shape
    V, d = table.shape
    Xt = X.T.astype(jnp.int32)
    Q = _make_gather(T, Bb, V)(table, Xt)
    return Q.transpose(2, 0, 1)
